# Initial kernel scaffold; baseline (speedup 1.0000x reference)
import jax
import jax.numpy as jnp
from jax.experimental import pallas as pl


def kernel(hidden_states, gate_w, gate_b):
    raise NotImplementedError("write your pallas kernel here")


# speedup vs baseline: 1.4245x; 1.4245x over previous
import functools

import jax
import jax.numpy as jnp
from jax import lax
from jax.experimental import pallas as pl
from jax.experimental.pallas import tpu as pltpu
from jax.experimental.pallas import tpu_sc as plsc

N_STATES = 12
T = 2048
B = 2
C = 1024
BC = B * C
TOP_K = 2


_TBLK = 512


def _gate_body(x_ref, w2_ref, out_ref, acc_ref):
    ti = pl.program_id(1)
    nt = pl.num_programs(1)
    s = jnp.sum(x_ref[0], axis=0, keepdims=True)

    @pl.when(ti == 0)
    def _init():
        acc_ref[...] = s

    @pl.when(ti > 0)
    def _acc():
        acc_ref[...] += s

    @pl.when(ti == nt - 1)
    def _fin():
        g = jnp.dot(acc_ref[...], w2_ref[...],
                    preferred_element_type=jnp.float32)
        out_ref[...] = g.reshape(1, 1, B)


_gate_call = pl.pallas_call(
    _gate_body,
    grid=(N_STATES, T // _TBLK),
    in_specs=[
        pl.BlockSpec((1, _TBLK, BC), lambda n, t: (n, t, 0)),
        pl.BlockSpec((BC, B), lambda n, t: (0, 0)),
    ],
    out_specs=pl.BlockSpec((1, 1, B), lambda n, t: (n, 0, 0)),
    out_shape=jax.ShapeDtypeStruct((N_STATES, 1, B), jnp.float32),
    scratch_shapes=[pltpu.VMEM((1, BC), jnp.float32)],
)


_NC = 2
_NS = 16
_NW = _NC * _NS
_NCHUNK = _NW // B
_TCHUNK = T // _NCHUNK
_TT = 32
_NIT = _TCHUNK // _TT
_L = 16

@functools.cache
def _build_combine():
    mesh = plsc.VectorSubcoreMesh(
        core_axis_name="c", subcore_axis_name="s",
        num_cores=_NC, num_subcores=_NS)
    return pl.kernel(
        _combine_body,
        out_type=jax.ShapeDtypeStruct((T, BC), jnp.float32),
        mesh=mesh,
        compiler_params=pltpu.CompilerParams(needs_layout_passes=False),
        scratch_types=[
            pltpu.VMEM((_TT,), jnp.int32),
            pltpu.VMEM((_TT,), jnp.int32),
            pltpu.VMEM((_TT, C), jnp.float32),
            pltpu.VMEM((_TT, C), jnp.float32),
            pltpu.VMEM((B, _L), jnp.float32),
            pltpu.SemaphoreType.DMA,
            pltpu.SemaphoreType.DMA,
        ],
    )


def _combine_body(hs_rows, gate_hbm, out_hbm, idx0_v, idx1_v, buf0, buf1,
                  gate_v, sem0, sem1):
    wid = lax.axis_index("s") * _NC + lax.axis_index("c")
    b = wid % B
    chunk = wid // B

    pltpu.sync_copy(gate_hbm, gate_v)
    g0 = gate_v[0, :]
    g1 = gate_v[1, :]
    bv = jnp.full((_L,), b, jnp.int32)
    gb = jnp.where(bv == 0, g0, g1)
    iota = lax.iota(jnp.int32, _L)
    m1 = jnp.max(gb)
    i1 = jnp.min(jnp.where(gb == m1, iota, 999))
    gb2 = jnp.where(iota == i1, -1e30, gb)
    m2 = jnp.max(gb2)
    i2 = jnp.min(jnp.where(gb2 == m2, iota, 999))
    e2 = jnp.exp(jnp.full((_L,), m2 - m1, jnp.float32))
    c1 = 0.5 / (1.0 + e2)
    c2 = e2 * c1

    for it in range(_NIT):
        t0 = chunk * _TCHUNK + it * _TT
        r0 = (i1 * T + t0 + iota) * B + b
        r1 = (i2 * T + t0 + iota) * B + b
        idx0_v[pl.ds(0, _L)] = r0
        idx0_v[pl.ds(_L, _L)] = r0 + _L * B
        idx1_v[pl.ds(0, _L)] = r1
        idx1_v[pl.ds(_L, _L)] = r1 + _L * B
        cp0 = pltpu.async_copy(hs_rows.at[idx0_v], buf0, sem0)
        cp1 = pltpu.async_copy(hs_rows.at[idx1_v], buf1, sem1)
        cp0.wait()
        cp1.wait()
        for r in range(_TT):
            def _col(j, carry, r=r):
                cc = j * _L
                v0 = buf0[r, pl.ds(cc, _L)]
                v1 = buf1[r, pl.ds(cc, _L)]
                buf0[r, pl.ds(cc, _L)] = c1 * v0 + c2 * v1
                return carry
            lax.fori_loop(0, C // _L, _col, 0)
        pltpu.sync_copy(buf0, out_hbm.at[pl.ds(t0, _TT), pl.ds(b * C, C)])


def kernel(hidden_states, gate_w, gate_b):
    n, t, b, c = hidden_states.shape
    assert (n, t, b, c) == (N_STATES, T, B, C)
    hs3 = hidden_states.reshape(n, t, b * c)
    w = gate_w.reshape(c) * (1.0 / t)
    w2 = jnp.zeros((b * c, b), jnp.float32)
    for bi in range(b):
        w2 = w2.at[bi * c:(bi + 1) * c, bi].set(w)
    gate_nb = _gate_call(hs3, w2)
    logits = gate_nb.reshape(n, b).T
    gate_pad = jnp.full((b, _L), -1e30, jnp.float32).at[:, :n].set(logits)
    out2 = _build_combine()(hidden_states.reshape(n * t * b, c), gate_pad)
    return out2.reshape(t, b, c)
